# baseline (speedup 1.0000x reference)
import functools

import jax
import jax.numpy as jnp
from jax import lax
from jax.experimental import pallas as pl
from jax.experimental.pallas import tpu as pltpu
from jax.experimental.pallas import tpu_sc as plsc

NC = 2
NS = 16
NW = NC * NS
L = 16

LN2 = 0.6931471805599453
CHUNK_W = 128


def _sqrt_vec(x):
    x = jnp.maximum(x, jnp.float32(1e-30))
    i = plsc.bitcast(x, jnp.int32)
    i = jnp.int32(0x5F3759DF) - (i >> 1)
    y = plsc.bitcast(i, jnp.float32)
    for _ in range(3):
        y = y * (jnp.float32(1.5) - jnp.float32(0.5) * x * y * y)
    return x * y


def _make_kernel(B, F):
    SPW = B // NW
    CHUNK = CHUNK_W
    NCHUNK = SPW // CHUNK
    GROUPS = CHUNK // L
    DEPTH = 2

    mesh = plsc.VectorSubcoreMesh(core_axis_name="c", subcore_axis_name="s")

    @functools.partial(
        pl.kernel,
        out_type=jax.ShapeDtypeStruct((NW, L), jnp.float32),
        mesh=mesh,
        compiler_params=pltpu.CompilerParams(needs_layout_passes=False),
        scratch_types=[
            pltpu.VMEM((NCHUNK, CHUNK), jnp.int32),
            [pltpu.VMEM((CHUNK, F // 2), jnp.float32)] * DEPTH,
            [pltpu.VMEM((CHUNK, F), jnp.float32)] * DEPTH,
            pltpu.VMEM((L,), jnp.float32),
            pltpu.VMEM((NCHUNK * GROUPS, L), jnp.float32),
            pltpu.SemaphoreType.DMA,
            [pltpu.SemaphoreType.DMA] * DEPTH,
            [pltpu.SemaphoreType.DMA] * DEPTH,
        ],
    )
    def closs_kernel(xs_hbm, ys_hbm, center_hbm, out_hbm,
                     idx_v, xbuf, rbuf, acc_v, res_v, si, sg, sx):
        cid = lax.axis_index("c")
        sid = lax.axis_index("s")
        wid = sid * NC + cid

        def start_xs(k):
            b = k % DEPTH
            return pltpu.async_copy(
                xs_hbm.at[pl.ds(wid * SPW + k * CHUNK, CHUNK)], xbuf[b], sx[b])

        def start_rows(k):
            b = k % DEPTH
            return pltpu.async_copy(center_hbm.at[idx_v.at[k]], rbuf[b], sg[b])

        xs_pending = {k: start_xs(k) for k in range(DEPTH - 1)}
        idx_cp = pltpu.async_copy(
            ys_hbm.at[pl.ds(wid * NCHUNK, NCHUNK)], idx_v, si)
        idx_cp.wait()
        row_pending = {k: start_rows(k) for k in range(DEPTH - 1)}

        lane = lax.iota(jnp.int32, L)

        for k in range(NCHUNK):
            b = k % DEPTH
            if k + DEPTH - 1 < NCHUNK:
                xs_pending[k + DEPTH - 1] = start_xs(k + DEPTH - 1)
                row_pending[k + DEPTH - 1] = start_rows(k + DEPTH - 1)
            row_pending.pop(k).wait()
            xs_pending.pop(k).wait()
            xs_v, rows_v = xbuf[b], rbuf[b]

            HW = F // 2

            @plsc.parallel_loop(0, GROUPS)
            def gbody(g, xs_v=xs_v, rows_v=rows_v, k=k):
                row = lane + g * jnp.int32(L)
                ss = [jnp.zeros((L,), jnp.float32) for _ in range(4)]
                colw = lane
                hvec = jnp.full((L,), HW, jnp.int32)
                hoff = jnp.full((L,), HW, jnp.int32)
                one = jnp.full((L,), 1, jnp.int32)
                for p in range(HW):
                    xw = plsc.load_gather(xs_v, [row, colw])
                    u0, u1 = plsc.unpack(plsc.bitcast(xw, jnp.bfloat16),
                                         format=plsc.PackFormat.INTERLEAVED)
                    cc0 = plsc.load_gather(rows_v, [row, colw])
                    cc1 = plsc.load_gather(rows_v, [row, colw + hoff])
                    ss[(2 * p) % 4] = ss[(2 * p) % 4] + jnp.exp(
                        (u0 - cc0) * jnp.float32(LN2))
                    ss[(2 * p + 1) % 4] = ss[(2 * p + 1) % 4] + jnp.exp(
                        (u1 - cc1) * jnp.float32(LN2))
                    if p + 1 < HW:
                        colw = colw + one
                        if p + 1 > HW - L:
                            colw = jnp.where(colw >= hvec, colw - hvec, colw)
                s = (ss[0] + ss[1]) + (ss[2] + ss[3])
                res_v[g + jnp.int32(k * GROUPS)] = _sqrt_vec(s)

        accs = [jnp.zeros((L,), jnp.float32) for _ in range(4)]
        for j in range(NCHUNK * GROUPS):
            accs[j % 4] = accs[j % 4] + res_v[j]
        acc_v[...] = (accs[0] + accs[1]) + (accs[2] + accs[3])
        pltpu.sync_copy(acc_v, out_hbm.at[wid])

    return closs_kernel


def kernel(xs, ys, center):
    B, F = xs.shape
    ys2d = ys.astype(jnp.int32).reshape(B // CHUNK_W, CHUNK_W)
    xb = xs.astype(jnp.bfloat16)
    xw = jax.lax.bitcast_convert_type(
        jnp.stack([xb[:, :F // 2], xb[:, F // 2:]], axis=-1), jnp.float32)
    partials = _make_kernel(B, F)(xw, ys2d, center)
    return jnp.sum(partials)


# speedup vs baseline: 1.2701x; 1.2701x over previous
import functools

import jax
import jax.numpy as jnp
from jax import lax
from jax.experimental import pallas as pl
from jax.experimental.pallas import tpu as pltpu
from jax.experimental.pallas import tpu_sc as plsc

NC = 2
NS = 16
NW = NC * NS
L = 16

LN2 = 0.6931471805599453
CHUNK_W = 128


def _sqrt_vec(x):
    x = jnp.maximum(x, jnp.float32(1e-30))
    i = plsc.bitcast(x, jnp.int32)
    i = jnp.int32(0x5F3759DF) - (i >> 1)
    y = plsc.bitcast(i, jnp.float32)
    for _ in range(3):
        y = y * (jnp.float32(1.5) - jnp.float32(0.5) * x * y * y)
    return x * y


def _make_kernel(B, F):
    SPW = B // NW
    CHUNK = CHUNK_W
    NCHUNK = SPW // CHUNK
    GROUPS = CHUNK // L
    DEPTH = 2

    mesh = plsc.VectorSubcoreMesh(core_axis_name="c", subcore_axis_name="s")

    @functools.partial(
        pl.kernel,
        out_type=jax.ShapeDtypeStruct((NW, L), jnp.float32),
        mesh=mesh,
        compiler_params=pltpu.CompilerParams(needs_layout_passes=False),
        scratch_types=[
            pltpu.VMEM((NCHUNK, CHUNK), jnp.int32),
            [pltpu.VMEM((CHUNK, F), jnp.float32)] * DEPTH,
            [pltpu.VMEM((CHUNK, F), jnp.float32)] * DEPTH,
            pltpu.VMEM((L,), jnp.float32),
            pltpu.VMEM((NCHUNK * GROUPS, L), jnp.float32),
            pltpu.SemaphoreType.DMA,
            [pltpu.SemaphoreType.DMA] * DEPTH,
            [pltpu.SemaphoreType.DMA] * DEPTH,
        ],
    )
    def closs_kernel(xs_hbm, ys_hbm, center_hbm, out_hbm,
                     idx_v, xbuf, rbuf, acc_v, res_v, si, sg, sx):
        cid = lax.axis_index("c")
        sid = lax.axis_index("s")
        wid = sid * NC + cid

        def start_xs(k):
            b = k % DEPTH
            return pltpu.async_copy(
                xs_hbm.at[pl.ds(wid * SPW + k * CHUNK, CHUNK)], xbuf[b], sx[b])

        def start_rows(k):
            b = k % DEPTH
            return pltpu.async_copy(center_hbm.at[idx_v.at[k]], rbuf[b], sg[b])

        xs_pending = {k: start_xs(k) for k in range(DEPTH - 1)}
        idx_cp = pltpu.async_copy(
            ys_hbm.at[pl.ds(wid * NCHUNK, NCHUNK)], idx_v, si)
        idx_cp.wait()
        row_pending = {k: start_rows(k) for k in range(DEPTH - 1)}

        lane = lax.iota(jnp.int32, L)

        for k in range(NCHUNK):
            b = k % DEPTH
            if k + DEPTH - 1 < NCHUNK:
                xs_pending[k + DEPTH - 1] = start_xs(k + DEPTH - 1)
                row_pending[k + DEPTH - 1] = start_rows(k + DEPTH - 1)
            row_pending.pop(k).wait()
            xs_pending.pop(k).wait()
            xs_v, rows_v = xbuf[b], rbuf[b]

            @plsc.parallel_loop(0, GROUPS)
            def gbody(g, xs_v=xs_v, rows_v=rows_v, k=k):
                row = lane + g * jnp.int32(L)
                ss = [jnp.zeros((L,), jnp.float32) for _ in range(4)]
                col = lane
                fvec = jnp.full((L,), F, jnp.int32)
                one = jnp.full((L,), 1, jnp.int32)
                for f in range(F):
                    xc = plsc.load_gather(xs_v, [row, col])
                    cc = plsc.load_gather(rows_v, [row, col])
                    ss[f % 4] = ss[f % 4] + jnp.exp((xc - cc) * jnp.float32(LN2))
                    if f + 1 < F:
                        col = col + one
                        if f + 1 > F - L:
                            col = jnp.where(col >= fvec, col - fvec, col)
                s = (ss[0] + ss[1]) + (ss[2] + ss[3])
                res_v[g + jnp.int32(k * GROUPS)] = _sqrt_vec(s)

        accs = [jnp.zeros((L,), jnp.float32) for _ in range(4)]
        for j in range(NCHUNK * GROUPS):
            accs[j % 4] = accs[j % 4] + res_v[j]
        acc_v[...] = (accs[0] + accs[1]) + (accs[2] + accs[3])
        pltpu.sync_copy(acc_v, out_hbm.at[wid])

    return closs_kernel


def kernel(xs, ys, center):
    B, F = xs.shape
    ys2d = ys.astype(jnp.int32).reshape(B // CHUNK_W, CHUNK_W)
    partials = _make_kernel(B, F)(xs, ys2d, center)
    return jnp.sum(partials)
